# baseline (speedup 1.0000x reference)
import numpy as np

import jax
import jax.numpy as jnp
from jax.experimental import pallas as pl
from jax.experimental.pallas import tpu as pltpu
from jax.sharding import Mesh, PartitionSpec as P

_TOKEN_BLOCK = 2048


def _moe_body(x_ref, gw_ref, gb_ref, ew_ref, eb_ref, o_ref, w_ref, xb_ref):
    e = pl.program_id(1)

    @pl.when(e == 0)
    def _gate():
        logits = jnp.dot(x_ref[...], gw_ref[...],
                         preferred_element_type=jnp.float32) + gb_ref[...]
        m = jnp.max(logits, axis=1, keepdims=True)
        p = jnp.exp(logits - m)
        w_ref[...] = p / jnp.sum(p, axis=1, keepdims=True)
        xb_ref[...] = x_ref[...].astype(jnp.bfloat16)

    lane = jax.lax.broadcasted_iota(jnp.int32, (1, w_ref.shape[1]), 1)
    w_e = jnp.sum(jnp.where(lane == e, w_ref[...], 0.0), axis=1, keepdims=True)

    acc = jnp.dot(xb_ref[...], ew_ref[0], preferred_element_type=jnp.float32)
    val = w_e * (acc + eb_ref[0])

    @pl.when(e == 0)
    def _init():
        o_ref[...] = val

    @pl.when(e > 0)
    def _accum():
        o_ref[...] += val


def _moe_one_device(x, gate_w, gate_b, expert_w, expert_b):
    tokens, f_in = x.shape
    n_exp, _, f_out = expert_w.shape
    bt = min(_TOKEN_BLOCK, tokens)
    grid = (tokens // bt, n_exp)

    return pl.pallas_call(
        _moe_body,
        grid=grid,
        in_specs=[
            pl.BlockSpec((bt, f_in), lambda i, e: (i, 0)),
            pl.BlockSpec((f_in, n_exp), lambda i, e: (0, 0)),
            pl.BlockSpec((1, n_exp), lambda i, e: (0, 0)),
            pl.BlockSpec((1, f_in, f_out), lambda i, e: (e, 0, 0)),
            pl.BlockSpec((1, 1, f_out), lambda i, e: (e, 0, 0)),
        ],
        out_specs=pl.BlockSpec((bt, f_out), lambda i, e: (i, 0)),
        out_shape=jax.ShapeDtypeStruct((tokens, f_out), jnp.float32),
        scratch_shapes=[pltpu.VMEM((bt, n_exp), jnp.float32),
                        pltpu.VMEM((bt, f_in), jnp.bfloat16)],
        compiler_params=pltpu.CompilerParams(
            dimension_semantics=("parallel", "arbitrary")),
    )(x, gate_w, gate_b, expert_w, expert_b)


def kernel(x, gate_w, gate_b, expert_w, expert_b):
    tokens, _ = x.shape
    n_exp, _, f_out = expert_w.shape
    gb2 = gate_b.reshape(1, n_exp)
    eb3 = expert_b.reshape(n_exp, 1, f_out)
    ewb = expert_w.astype(jnp.bfloat16)

    devs = jax.devices()
    n_dev = 2 if (len(devs) >= 2 and tokens % (2 * 8) == 0) else 1
    if n_dev == 1:
        return _moe_one_device(x, gate_w, gb2, ewb, eb3)

    mesh = Mesh(np.array(devs[:n_dev]), ("d",))
    f = jax.shard_map(
        _moe_one_device, mesh=mesh,
        in_specs=(P("d", None), P(None, None), P(None, None),
                  P(None, None, None), P(None, None, None)),
        out_specs=P("d", None), check_vma=False)
    return f(x, gate_w, gb2, ewb, eb3)


# speedup vs baseline: 1.0147x; 1.0147x over previous
import numpy as np

import jax
import jax.numpy as jnp
from jax.experimental import pallas as pl
from jax.experimental.pallas import tpu as pltpu
from jax.sharding import Mesh, PartitionSpec as P

_TOKEN_BLOCK = 2048


def _moe_body(x_ref, gw_ref, gb_ref, ew_ref, eb_ref, o_ref, w_ref):
    e = pl.program_id(1)

    @pl.when(e == 0)
    def _gate():
        logits = jnp.dot(x_ref[...], gw_ref[...],
                         preferred_element_type=jnp.float32) + gb_ref[...]
        m = jnp.max(logits, axis=1, keepdims=True)
        p = jnp.exp(logits - m)
        w_ref[...] = p / jnp.sum(p, axis=1, keepdims=True)

    lane = jax.lax.broadcasted_iota(jnp.int32, (1, w_ref.shape[1]), 1)
    w_e = jnp.sum(jnp.where(lane == e, w_ref[...], 0.0), axis=1, keepdims=True)

    acc = jnp.dot(x_ref[...], ew_ref[0], preferred_element_type=jnp.float32)
    val = w_e * (acc + eb_ref[0])

    @pl.when(e == 0)
    def _init():
        o_ref[...] = val

    @pl.when(e > 0)
    def _accum():
        o_ref[...] += val


def _moe_one_device(x, gate_w, gate_b, expert_w, expert_b):
    tokens, f_in = x.shape
    n_exp, _, f_out = expert_w.shape
    bt = min(_TOKEN_BLOCK, tokens)
    grid = (tokens // bt, n_exp)

    return pl.pallas_call(
        _moe_body,
        grid=grid,
        in_specs=[
            pl.BlockSpec((bt, f_in), lambda i, e: (i, 0)),
            pl.BlockSpec((f_in, n_exp), lambda i, e: (0, 0)),
            pl.BlockSpec((1, n_exp), lambda i, e: (0, 0)),
            pl.BlockSpec((1, f_in, f_out), lambda i, e: (e, 0, 0)),
            pl.BlockSpec((1, 1, f_out), lambda i, e: (e, 0, 0)),
        ],
        out_specs=pl.BlockSpec((bt, f_out), lambda i, e: (i, 0)),
        out_shape=jax.ShapeDtypeStruct((tokens, f_out), jnp.float32),
        scratch_shapes=[pltpu.VMEM((bt, n_exp), jnp.float32)],
        compiler_params=pltpu.CompilerParams(
            dimension_semantics=("parallel", "arbitrary")),
    )(x, gate_w, gate_b, expert_w, expert_b)


def kernel(x, gate_w, gate_b, expert_w, expert_b):
    tokens, _ = x.shape
    n_exp, _, f_out = expert_w.shape
    gb2 = gate_b.reshape(1, n_exp)
    eb3 = expert_b.reshape(n_exp, 1, f_out)
    devs = jax.devices()
    n_dev = 2 if (len(devs) >= 2 and tokens % (2 * 8) == 0) else 1
    if n_dev == 1:
        return _moe_one_device(x, gate_w, gb2, expert_w, eb3)

    mesh = Mesh(np.array(devs[:n_dev]), ("d",))
    f = jax.shard_map(
        _moe_one_device, mesh=mesh,
        in_specs=(P("d", None), P(None, None), P(None, None),
                  P(None, None, None), P(None, None, None)),
        out_specs=P("d", None), check_vma=False)
    return f(x, gate_w, gb2, expert_w, eb3)


# speedup vs baseline: 1.0429x; 1.0278x over previous
import numpy as np

import jax
import jax.numpy as jnp
from jax.experimental import pallas as pl
from jax.experimental.pallas import tpu as pltpu
from jax.sharding import Mesh, PartitionSpec as P

_TOKEN_BLOCK = 2048


def _moe_body(x_ref, gw_ref, gb_ref, ew_ref, eb_ref, o_ref, w_ref):
    e = pl.program_id(1)

    @pl.when(e == 0)
    def _gate():
        logits = jnp.dot(x_ref[...], gw_ref[...],
                         preferred_element_type=jnp.float32) + gb_ref[...]
        m = jnp.max(logits, axis=1, keepdims=True)
        p = jnp.exp(logits - m)
        w_ref[...] = p / jnp.sum(p, axis=1, keepdims=True)

    lane = jax.lax.broadcasted_iota(jnp.int32, (1, w_ref.shape[1]), 1)
    w_e = jnp.sum(jnp.where(lane == e, w_ref[...], 0.0), axis=1, keepdims=True)

    acc = jnp.dot(x_ref[...], ew_ref[0], preferred_element_type=jnp.float32)
    val = w_e * (acc + eb_ref[0])

    @pl.when(e == 0)
    def _init():
        o_ref[...] = val

    @pl.when(e > 0)
    def _accum():
        o_ref[...] += val


def _moe_one_device(x, gate_w, gate_b, expert_w, expert_b):
    tokens, f_in = x.shape
    n_exp, _, f_out = expert_w.shape
    gate_b = gate_b.reshape(1, n_exp)
    expert_b = expert_b.reshape(n_exp, 1, f_out)
    bt = min(_TOKEN_BLOCK, tokens)
    grid = (tokens // bt, n_exp)

    return pl.pallas_call(
        _moe_body,
        grid=grid,
        in_specs=[
            pl.BlockSpec((bt, f_in), lambda i, e: (i, 0)),
            pl.BlockSpec((f_in, n_exp), lambda i, e: (0, 0)),
            pl.BlockSpec((1, n_exp), lambda i, e: (0, 0)),
            pl.BlockSpec((1, f_in, f_out), lambda i, e: (e, 0, 0)),
            pl.BlockSpec((1, 1, f_out), lambda i, e: (e, 0, 0)),
        ],
        out_specs=pl.BlockSpec((bt, f_out), lambda i, e: (i, 0)),
        out_shape=jax.ShapeDtypeStruct((tokens, f_out), jnp.float32),
        scratch_shapes=[pltpu.VMEM((bt, n_exp), jnp.float32)],
        compiler_params=pltpu.CompilerParams(
            dimension_semantics=("parallel", "arbitrary")),
    )(x, gate_w, gate_b, expert_w, expert_b)


def kernel(x, gate_w, gate_b, expert_w, expert_b):
    tokens, _ = x.shape
    devs = jax.devices()
    n_dev = 2 if (len(devs) >= 2 and tokens % (2 * 8) == 0) else 1
    if n_dev == 1:
        return _moe_one_device(x, gate_w, gate_b, expert_w, expert_b)

    mesh = Mesh(np.array(devs[:n_dev]), ("d",))
    specs = (P("d", None), P(None, None), P(None,),
             P(None, None, None), P(None, None))
    rep = [jax.lax.with_sharding_constraint(a, jax.NamedSharding(mesh, s))
           for a, s in zip((x, gate_w, gate_b, expert_w, expert_b), specs)]
    f = jax.shard_map(
        _moe_one_device, mesh=mesh, in_specs=specs,
        out_specs=P("d", None), check_vma=False)
    return f(*rep)
